# Initial kernel scaffold; baseline (speedup 1.0000x reference)
import functools

import numpy as np
import jax
import jax.numpy as jnp
from jax import lax
from jax.experimental import pallas as pl
from jax.experimental.pallas import tpu as pltpu
from jax.experimental.pallas import tpu_sc as plsc

DIM = 128
N_NODES = 10000
N_EDGES = 320000
N_CONV = 3
CUTOFF = 5.0
N_CENTERS = 5
_CENTERS = [float(c) for c in np.linspace(0.0, CUTOFF, N_CENTERS)]
GAP = _CENTERS[1] - _CENTERS[0]
TYPE_NUM = 100

NC, NS = 2, 16
NW = NC * NS
EPW = N_EDGES // NW
EB = 80
NCHUNK = EPW // EB
RPS = N_NODES // NS
_ROW_CHUNKS = [(o, min(EB, RPS - o)) for o in range(0, RPS, EB)]

EB_H = 8000


def _sp05(x):
    z = 0.5 * x
    return jnp.where(z > 14.0, x, 2.0 * jnp.logaddexp(0.0, z))


def _shift_sp(x):
    return jnp.where(x > 20.0, x, jnp.logaddexp(0.0, x)) - float(np.log(2.0))


def _embed_body(nt_ref, emb_ref, out_ref):
    nt = nt_ref[...]
    ids = lax.broadcasted_iota(jnp.int32, (1, TYPE_NUM), 1)
    onehot = (nt == ids).astype(jnp.float32)
    out_ref[...] = jnp.dot(onehot, emb_ref[...],
                           preferred_element_type=jnp.float32)


def _embed(node_type, emb):
    return pl.pallas_call(
        _embed_body,
        out_shape=jax.ShapeDtypeStruct((N_NODES, DIM), jnp.float32),
    )(node_type.reshape(N_NODES, 1), emb)


def _h_body(d_ref, w1t_ref, b1_ref, w2t_ref, b2_ref, out_ref):
    d = d_ref[...]
    acc = jnp.broadcast_to(b1_ref[...], (EB_H, DIM))
    for k in range(N_CENTERS):
        r = jnp.exp((-1.0 / GAP) * (d - _CENTERS[k]) ** 2)
        acc = acc + r * w1t_ref[k][None, :]
    h = _sp05(acc)
    h = jnp.dot(h, w2t_ref[...], preferred_element_type=jnp.float32) + b2_ref[...]
    out_ref[...] = _sp05(h)


def _edge_filter(dist2, wc1, bc1, wc2, bc2):
    grid = (N_EDGES // EB_H,)
    return pl.pallas_call(
        _h_body,
        grid=grid,
        in_specs=[
            pl.BlockSpec((EB_H, 1), lambda i: (i, 0)),
            pl.BlockSpec((N_CENTERS, DIM), lambda i: (0, 0)),
            pl.BlockSpec((1, DIM), lambda i: (0, 0)),
            pl.BlockSpec((DIM, DIM), lambda i: (0, 0)),
            pl.BlockSpec((1, DIM), lambda i: (0, 0)),
        ],
        out_specs=pl.BlockSpec((EB_H, DIM), lambda i: (i, 0)),
        out_shape=jax.ShapeDtypeStruct((N_EDGES, DIM), jnp.float32),
    )(dist2, wc1.T, bc1.reshape(1, DIM), wc2.T, bc2.reshape(1, DIM))


def _mm_body(x_ref, wt_ref, out_ref):
    out_ref[...] = jnp.dot(x_ref[...], wt_ref[...],
                           preferred_element_type=jnp.float32)


def _new_node(node, w1):
    return pl.pallas_call(
        _mm_body,
        out_shape=jax.ShapeDtypeStruct((N_NODES, DIM), jnp.float32),
    )(node, w1.T)


def _update_body(node_ref, agg_ref, w2t_ref, b2_ref, w3t_ref, b3_ref, out_ref):
    agg = agg_ref[0] + agg_ref[1]
    cf1 = _sp05(jnp.dot(agg, w2t_ref[...],
                        preferred_element_type=jnp.float32) + b2_ref[...])
    out_ref[...] = (node_ref[...]
                    + jnp.dot(cf1, w3t_ref[...],
                              preferred_element_type=jnp.float32)
                    + b3_ref[...])


def _update(node, agg2, w2, b2, w3, b3):
    return pl.pallas_call(
        _update_body,
        out_shape=jax.ShapeDtypeStruct((N_NODES, DIM), jnp.float32),
    )(node, agg2, w2.T, b2.reshape(1, DIM), w3.T, b3.reshape(1, DIM))


def _head_body(node_ref, wa1t_ref, ba1_ref, wclst_ref, bcls_ref, out_ref):
    atom = jnp.dot(node_ref[...], wa1t_ref[...],
                   preferred_element_type=jnp.float32) + ba1_ref[...]
    res = _shift_sp(atom)
    out_ref[...] = jnp.dot(res, wclst_ref[...],
                           preferred_element_type=jnp.float32) + bcls_ref[...]


def _head(node, wa1, ba1, wcls, bcls):
    return pl.pallas_call(
        _head_body,
        out_shape=jax.ShapeDtypeStruct((N_NODES, TYPE_NUM), jnp.float32),
    )(node, wa1.T, ba1.reshape(1, 64), wcls.T, bcls.reshape(1, TYPE_NUM))


def _sc_body(nn_hbm, h_hbm, src_hbm, dst_hbm, out_hbm,
             srcv, dstv, gv, hv, aggs, sem):
    cid = lax.axis_index("c")
    sid = lax.axis_index("s")
    wid = cid * NS + sid

    zvec = jnp.zeros((16,), jnp.float32)

    def zrow(r, _):
        for j in range(DIM // 16):
            gv[r, pl.ds(j * 16, 16)] = zvec
        return 0

    lax.fori_loop(0, EB, zrow, 0)
    row0 = sid * RPS
    for off, n in _ROW_CHUNKS:
        pltpu.sync_copy(gv.at[pl.ds(0, n)], aggs.at[pl.ds(row0 + off, n)])
    plsc.subcore_barrier()

    base_e = wid * EPW

    def chunk(c, _):
        eb = base_e + c * EB
        pltpu.sync_copy(src_hbm.at[pl.ds(eb, EB)], srcv.at[0])
        pltpu.sync_copy(dst_hbm.at[pl.ds(eb, EB)], dstv.at[0])
        pltpu.sync_copy(h_hbm.at[pl.ds(eb, EB)], hv)
        pltpu.async_copy(nn_hbm.at[srcv.at[0]], gv, sem).wait()

        def mulrow(r, _):
            for j in range(DIM // 16):
                sl = pl.ds(j * 16, 16)
                gv[r, sl] = gv[r, sl] * hv[r, sl]
            return 0

        lax.fori_loop(0, EB, mulrow, 0)
        pltpu.sync_copy(gv, aggs.at[dstv.at[0]], add=True)
        return 0

    lax.fori_loop(0, NCHUNK, chunk, 0)
    plsc.subcore_barrier()

    out_row0 = cid * N_NODES + row0
    for off, n in _ROW_CHUNKS:
        pltpu.sync_copy(aggs.at[pl.ds(row0 + off, n)], gv.at[pl.ds(0, n)])
        pltpu.sync_copy(gv.at[pl.ds(0, n)], out_hbm.at[pl.ds(out_row0 + off, n)])


def _segment_msg_sum(nn, h, src, dst):
    mesh = plsc.VectorSubcoreMesh(core_axis_name="c", subcore_axis_name="s")
    f = pl.kernel(
        _sc_body,
        out_type=jax.ShapeDtypeStruct((NC * N_NODES, DIM), jnp.float32),
        mesh=mesh,
        scratch_types=[
            pltpu.VMEM((1, EB), jnp.int32),
            pltpu.VMEM((1, EB), jnp.int32),
            pltpu.VMEM((EB, DIM), jnp.float32),
            pltpu.VMEM((EB, DIM), jnp.float32),
            pltpu.VMEM_SHARED((N_NODES, DIM), jnp.float32),
            pltpu.SemaphoreType.DMA,
        ],
    )
    out = f(nn, h, src, dst)
    return out.reshape(NC, N_NODES, DIM)


def kernel(node_type, edge_index, distance, emb, W1s, Wc1s, bc1s, Wc2s, bc2s,
           W2s, b2s, W3s, b3s, Wa1, ba1, Wcls, bcls):
    src = edge_index[0]
    dst = edge_index[1]
    dist2 = distance.reshape(N_EDGES, 1)

    node = _embed(node_type, emb)
    for i in range(N_CONV):
        h = _edge_filter(dist2, Wc1s[i], bc1s[i], Wc2s[i], bc2s[i])
        nn = _new_node(node, W1s[i])
        agg2 = _segment_msg_sum(nn, h, src, dst)
        node = _update(node, agg2, W2s[i], b2s[i], W3s[i], b3s[i])
    return _head(node, Wa1, ba1, Wcls, bcls)


# speedup vs baseline: 2.3435x; 2.3435x over previous
import functools

import numpy as np
import jax
import jax.numpy as jnp
from jax import lax
from jax.experimental import pallas as pl
from jax.experimental.pallas import tpu as pltpu
from jax.experimental.pallas import tpu_sc as plsc

DIM = 128
N_NODES = 10000
N_EDGES = 320000
N_CONV = 3
CUTOFF = 5.0
N_CENTERS = 5
_CENTERS = [float(c) for c in np.linspace(0.0, CUTOFF, N_CENTERS)]
GAP = _CENTERS[1] - _CENTERS[0]
TYPE_NUM = 100

NC, NS = 2, 16
NW = NC * NS
EPW = N_EDGES // NW
EB = 80
NCHUNK = EPW // EB
RPS = 624
_ROW_CHUNKS = [(o, min(EB, RPS - o)) for o in range(0, RPS, EB)]
_TAIL_ROW0 = NS * RPS
_TAIL_N = N_NODES - _TAIL_ROW0

EB_H = 8000


def _sp05(x):
    z = 0.5 * x
    return jnp.where(z > 14.0, x, 2.0 * jnp.logaddexp(0.0, z))


def _shift_sp(x):
    return jnp.where(x > 20.0, x, jnp.logaddexp(0.0, x)) - float(np.log(2.0))


def _embed_body(nt_ref, emb_ref, out_ref):
    nt = nt_ref[...]
    ids = lax.broadcasted_iota(jnp.int32, (1, TYPE_NUM), 1)
    onehot = (nt == ids).astype(jnp.float32)
    out_ref[...] = jnp.dot(onehot, emb_ref[...],
                           preferred_element_type=jnp.float32)


def _embed(node_type, emb):
    return pl.pallas_call(
        _embed_body,
        out_shape=jax.ShapeDtypeStruct((N_NODES, DIM), jnp.float32),
    )(node_type.reshape(N_NODES, 1), emb)


def _h_body(d_ref, w1t_ref, b1_ref, w2t_ref, b2_ref, out_ref):
    d = d_ref[...]
    acc = jnp.broadcast_to(b1_ref[...], (EB_H, DIM))
    for k in range(N_CENTERS):
        r = jnp.exp((-1.0 / GAP) * (d - _CENTERS[k]) ** 2)
        acc = acc + r * w1t_ref[k][None, :]
    h = _sp05(acc)
    h = jnp.dot(h, w2t_ref[...], preferred_element_type=jnp.float32) + b2_ref[...]
    out_ref[...] = _sp05(h)


def _edge_filter(dist2, wc1, bc1, wc2, bc2):
    grid = (N_EDGES // EB_H,)
    return pl.pallas_call(
        _h_body,
        grid=grid,
        in_specs=[
            pl.BlockSpec((EB_H, 1), lambda i: (i, 0)),
            pl.BlockSpec((N_CENTERS, DIM), lambda i: (0, 0)),
            pl.BlockSpec((1, DIM), lambda i: (0, 0)),
            pl.BlockSpec((DIM, DIM), lambda i: (0, 0)),
            pl.BlockSpec((1, DIM), lambda i: (0, 0)),
        ],
        out_specs=pl.BlockSpec((EB_H, DIM), lambda i: (i, 0)),
        out_shape=jax.ShapeDtypeStruct((N_EDGES, DIM), jnp.float32),
    )(dist2, wc1.T, bc1.reshape(1, DIM), wc2.T, bc2.reshape(1, DIM))


def _mm_body(x_ref, wt_ref, out_ref):
    out_ref[...] = jnp.dot(x_ref[...], wt_ref[...],
                           preferred_element_type=jnp.float32)


def _new_node(node, w1):
    return pl.pallas_call(
        _mm_body,
        out_shape=jax.ShapeDtypeStruct((N_NODES, DIM), jnp.float32),
    )(node, w1.T)


def _update_body(node_ref, agg_ref, w2t_ref, b2_ref, w3t_ref, b3_ref, out_ref):
    agg = agg_ref[0] + agg_ref[1]
    cf1 = _sp05(jnp.dot(agg, w2t_ref[...],
                        preferred_element_type=jnp.float32) + b2_ref[...])
    out_ref[...] = (node_ref[...]
                    + jnp.dot(cf1, w3t_ref[...],
                              preferred_element_type=jnp.float32)
                    + b3_ref[...])


def _update(node, agg2, w2, b2, w3, b3):
    return pl.pallas_call(
        _update_body,
        out_shape=jax.ShapeDtypeStruct((N_NODES, DIM), jnp.float32),
    )(node, agg2, w2.T, b2.reshape(1, DIM), w3.T, b3.reshape(1, DIM))


def _head_body(node_ref, wa1t_ref, ba1_ref, wclst_ref, bcls_ref, out_ref):
    atom = jnp.dot(node_ref[...], wa1t_ref[...],
                   preferred_element_type=jnp.float32) + ba1_ref[...]
    res = _shift_sp(atom)
    out_ref[...] = jnp.dot(res, wclst_ref[...],
                           preferred_element_type=jnp.float32) + bcls_ref[...]


def _head(node, wa1, ba1, wcls, bcls):
    return pl.pallas_call(
        _head_body,
        out_shape=jax.ShapeDtypeStruct((N_NODES, TYPE_NUM), jnp.float32),
    )(node, wa1.T, ba1.reshape(1, 64), wcls.T, bcls.reshape(1, TYPE_NUM))


def _sc_body(nn_hbm, h_hbm, src_hbm, dst_hbm, out_hbm,
             srcv, dstv, gv, hv, aggs, sem):
    cid = lax.axis_index("c")
    sid = lax.axis_index("s")
    wid = cid * NS + sid

    zvec = jnp.zeros((16,), jnp.float32)

    def zrow(r, _):
        for j in range(DIM // 16):
            gv[r, pl.ds(j * 16, 16)] = zvec
        return 0

    lax.fori_loop(0, EB, zrow, 0)
    row0 = sid * RPS
    for off, n in _ROW_CHUNKS:
        pltpu.sync_copy(gv.at[pl.ds(0, n)], aggs.at[pl.ds(row0 + off, n)])

    @pl.when(sid == NS - 1)
    def _():
        pltpu.sync_copy(gv.at[pl.ds(0, _TAIL_N)],
                        aggs.at[pl.ds(_TAIL_ROW0, _TAIL_N)])

    plsc.subcore_barrier()

    base_e = wid * EPW

    def chunk(c, _):
        eb = base_e + c * EB
        pltpu.sync_copy(src_hbm.at[pl.ds(eb, EB)], srcv.at[0])
        pltpu.sync_copy(dst_hbm.at[pl.ds(eb, EB)], dstv.at[0])
        pltpu.sync_copy(h_hbm.at[pl.ds(eb, EB)], hv)
        pltpu.async_copy(nn_hbm.at[srcv.at[0]], gv, sem).wait()

        def mulrow(r, _):
            for j in range(DIM // 16):
                sl = pl.ds(j * 16, 16)
                gv[r, sl] = gv[r, sl] * hv[r, sl]
            return 0

        lax.fori_loop(0, EB, mulrow, 0)
        pltpu.sync_copy(gv, aggs.at[dstv.at[0]], add=True)
        return 0

    lax.fori_loop(0, NCHUNK, chunk, 0)
    plsc.subcore_barrier()

    out_row0 = cid * N_NODES + row0
    for off, n in _ROW_CHUNKS:
        pltpu.sync_copy(aggs.at[pl.ds(row0 + off, n)], gv.at[pl.ds(0, n)])
        pltpu.sync_copy(gv.at[pl.ds(0, n)], out_hbm.at[pl.ds(out_row0 + off, n)])

    @pl.when(sid == NS - 1)
    def _():
        pltpu.sync_copy(aggs.at[pl.ds(_TAIL_ROW0, _TAIL_N)],
                        gv.at[pl.ds(0, _TAIL_N)])
        pltpu.sync_copy(gv.at[pl.ds(0, _TAIL_N)],
                        out_hbm.at[pl.ds(cid * N_NODES + _TAIL_ROW0, _TAIL_N)])


def _segment_msg_sum(nn, h, src, dst):
    mesh = plsc.VectorSubcoreMesh(core_axis_name="c", subcore_axis_name="s")
    f = pl.kernel(
        _sc_body,
        out_type=jax.ShapeDtypeStruct((NC * N_NODES, DIM), jnp.float32),
        mesh=mesh,
        scratch_types=[
            pltpu.VMEM((1, EB), jnp.int32),
            pltpu.VMEM((1, EB), jnp.int32),
            pltpu.VMEM((EB, DIM), jnp.float32),
            pltpu.VMEM((EB, DIM), jnp.float32),
            pltpu.VMEM_SHARED((N_NODES, DIM), jnp.float32),
            pltpu.SemaphoreType.DMA,
        ],
    )
    out = f(nn, h, src, dst)
    return out.reshape(NC, N_NODES, DIM)


def kernel(node_type, edge_index, distance, emb, W1s, Wc1s, bc1s, Wc2s, bc2s,
           W2s, b2s, W3s, b3s, Wa1, ba1, Wcls, bcls):
    src = edge_index[0]
    dst = edge_index[1]
    dist2 = distance.reshape(N_EDGES, 1)

    node = _embed(node_type, emb)
    for i in range(N_CONV):
        h = _edge_filter(dist2, Wc1s[i], bc1s[i], Wc2s[i], bc2s[i])
        nn = _new_node(node, W1s[i])
        agg2 = _segment_msg_sum(nn, h, src, dst)
        node = _update(node, agg2, W2s[i], b2s[i], W3s[i], b3s[i])
    return _head(node, Wa1, ba1, Wcls, bcls)


# speedup vs baseline: 2.5851x; 1.1031x over previous
import functools

import numpy as np
import jax
import jax.numpy as jnp
from jax import lax
from jax.experimental import pallas as pl
from jax.experimental.pallas import tpu as pltpu
from jax.experimental.pallas import tpu_sc as plsc

DIM = 128
N_NODES = 10000
N_EDGES = 320000
N_CONV = 3
CUTOFF = 5.0
N_CENTERS = 5
_CENTERS = [float(c) for c in np.linspace(0.0, CUTOFF, N_CENTERS)]
GAP = _CENTERS[1] - _CENTERS[0]
TYPE_NUM = 100

NC, NS = 2, 16
NW = NC * NS
EPW = N_EDGES // NW
EB = 16
NCHUNK = EPW // EB
RPS = 624
_ROW_CHUNKS = [(o, min(EB, RPS - o)) for o in range(0, RPS, EB)]
_TAIL_ROW0 = NS * RPS
_TAIL_N = N_NODES - _TAIL_ROW0

EB_H = 8000


def _sp05(x):
    z = 0.5 * x
    return jnp.where(z > 14.0, x, 2.0 * jnp.logaddexp(0.0, z))


def _shift_sp(x):
    return jnp.where(x > 20.0, x, jnp.logaddexp(0.0, x)) - float(np.log(2.0))


def _embed_body(nt_ref, emb_ref, out_ref):
    nt = nt_ref[...]
    ids = lax.broadcasted_iota(jnp.int32, (1, TYPE_NUM), 1)
    onehot = (nt == ids).astype(jnp.float32)
    out_ref[...] = jnp.dot(onehot, emb_ref[...],
                           preferred_element_type=jnp.float32)


def _embed(node_type, emb):
    return pl.pallas_call(
        _embed_body,
        out_shape=jax.ShapeDtypeStruct((N_NODES, DIM), jnp.float32),
    )(node_type.reshape(N_NODES, 1), emb)


def _h_body(d_ref, w1t_ref, b1_ref, w2t_ref, b2_ref, out_ref):
    d = d_ref[...]
    acc = jnp.broadcast_to(b1_ref[...], (EB_H, DIM))
    for k in range(N_CENTERS):
        r = jnp.exp((-1.0 / GAP) * (d - _CENTERS[k]) ** 2)
        acc = acc + r * w1t_ref[k][None, :]
    h = _sp05(acc)
    h = jnp.dot(h, w2t_ref[...], preferred_element_type=jnp.float32) + b2_ref[...]
    out_ref[...] = _sp05(h)


def _edge_filter(dist2, wc1, bc1, wc2, bc2):
    grid = (N_EDGES // EB_H,)
    return pl.pallas_call(
        _h_body,
        grid=grid,
        in_specs=[
            pl.BlockSpec((EB_H, 1), lambda i: (i, 0)),
            pl.BlockSpec((N_CENTERS, DIM), lambda i: (0, 0)),
            pl.BlockSpec((1, DIM), lambda i: (0, 0)),
            pl.BlockSpec((DIM, DIM), lambda i: (0, 0)),
            pl.BlockSpec((1, DIM), lambda i: (0, 0)),
        ],
        out_specs=pl.BlockSpec((EB_H, DIM), lambda i: (i, 0)),
        out_shape=jax.ShapeDtypeStruct((N_EDGES, DIM), jnp.float32),
    )(dist2, wc1.T, bc1.reshape(1, DIM), wc2.T, bc2.reshape(1, DIM))


def _mm_body(x_ref, wt_ref, out_ref):
    out_ref[...] = jnp.dot(x_ref[...], wt_ref[...],
                           preferred_element_type=jnp.float32)


def _new_node(node, w1):
    return pl.pallas_call(
        _mm_body,
        out_shape=jax.ShapeDtypeStruct((N_NODES, DIM), jnp.float32),
    )(node, w1.T)


def _update_body(node_ref, agg_ref, w2t_ref, b2_ref, w3t_ref, b3_ref, out_ref):
    agg = agg_ref[0] + agg_ref[1]
    cf1 = _sp05(jnp.dot(agg, w2t_ref[...],
                        preferred_element_type=jnp.float32) + b2_ref[...])
    out_ref[...] = (node_ref[...]
                    + jnp.dot(cf1, w3t_ref[...],
                              preferred_element_type=jnp.float32)
                    + b3_ref[...])


def _update(node, agg2, w2, b2, w3, b3):
    return pl.pallas_call(
        _update_body,
        out_shape=jax.ShapeDtypeStruct((N_NODES, DIM), jnp.float32),
    )(node, agg2, w2.T, b2.reshape(1, DIM), w3.T, b3.reshape(1, DIM))


def _head_body(node_ref, wa1t_ref, ba1_ref, wclst_ref, bcls_ref, out_ref):
    atom = jnp.dot(node_ref[...], wa1t_ref[...],
                   preferred_element_type=jnp.float32) + ba1_ref[...]
    res = _shift_sp(atom)
    out_ref[...] = jnp.dot(res, wclst_ref[...],
                           preferred_element_type=jnp.float32) + bcls_ref[...]


def _head(node, wa1, ba1, wcls, bcls):
    return pl.pallas_call(
        _head_body,
        out_shape=jax.ShapeDtypeStruct((N_NODES, TYPE_NUM), jnp.float32),
    )(node, wa1.T, ba1.reshape(1, 64), wcls.T, bcls.reshape(1, TYPE_NUM))


NBUF = 5
NGROUP = NCHUNK // NBUF
WB = 64
_WB_CHUNKS = [(o, min(WB, RPS - o)) for o in range(0, RPS, WB)]


def _sc_body(nn_hbm, h_hbm, src_hbm, dst_hbm, out_hbm,
             isrc, idst, wb, *rest):
    gvs = rest[0:NBUF]
    hvs = rest[NBUF:2 * NBUF]
    aggs = rest[2 * NBUF]
    hsems = rest[2 * NBUF + 1:2 * NBUF + 1 + NBUF]
    gsems = rest[2 * NBUF + 1 + NBUF:2 * NBUF + 1 + 2 * NBUF]
    ssems = rest[2 * NBUF + 1 + 2 * NBUF:2 * NBUF + 1 + 3 * NBUF]

    cid = lax.axis_index("c")
    sid = lax.axis_index("s")
    wid = cid * NS + sid
    base_e = wid * EPW

    pltpu.sync_copy(src_hbm.at[pl.ds(base_e, EPW)], isrc)
    pltpu.sync_copy(dst_hbm.at[pl.ds(base_e, EPW)], idst)

    zvec = jnp.zeros((16,), jnp.float32)

    def zrow(r, _):
        for j in range(DIM // 16):
            wb[r, pl.ds(j * 16, 16)] = zvec
        return 0

    lax.fori_loop(0, WB, zrow, 0)
    row0 = sid * RPS
    for off, n in _WB_CHUNKS:
        pltpu.sync_copy(wb.at[pl.ds(0, n)], aggs.at[pl.ds(row0 + off, n)])

    @pl.when(sid == NS - 1)
    def _():
        pltpu.sync_copy(wb.at[pl.ds(0, _TAIL_N)],
                        aggs.at[pl.ds(_TAIL_ROW0, _TAIL_N)])

    plsc.subcore_barrier()

    def group(g, _):
        c0 = g * NBUF
        in_descs = []
        for b in range(NBUF):
            c = c0 + b
            e0 = base_e + c * EB

            @pl.when(g > 0)
            def _(b=b, c=c):
                dreg = idst[pl.ds(c * EB, EB)]
                pltpu.make_async_copy(gvs[b], aggs.at[dreg], ssems[b]).wait()

            sreg = isrc[pl.ds(c * EB, EB)]
            hd = pltpu.async_copy(h_hbm.at[pl.ds(e0, EB)], hvs[b], hsems[b])
            gd = pltpu.async_copy(nn_hbm.at[sreg], gvs[b], gsems[b])
            in_descs.append((hd, gd))

        for b in range(NBUF):
            c = c0 + b
            hd, gd = in_descs[b]
            hd.wait()
            gd.wait()
            gvb, hvb = gvs[b], hvs[b]

            def mulrow(r, _):
                for j in range(DIM // 16):
                    sl = pl.ds(j * 16, 16)
                    gvb[r, sl] = gvb[r, sl] * hvb[r, sl]
                return 0

            lax.fori_loop(0, EB, mulrow, 0, unroll=8)
            dreg = idst[pl.ds(c * EB, EB)]
            pltpu.async_copy(gvb, aggs.at[dreg], ssems[b], add=True)
        return 0

    lax.fori_loop(0, NGROUP, group, 0)
    for b in range(NBUF):
        dreg = idst[pl.ds(((NGROUP - 1) * NBUF + b) * EB, EB)]
        pltpu.make_async_copy(gvs[b], aggs.at[dreg], ssems[b]).wait()
    plsc.subcore_barrier()

    out_row0 = cid * N_NODES + row0
    for off, n in _WB_CHUNKS:
        pltpu.sync_copy(aggs.at[pl.ds(row0 + off, n)], wb.at[pl.ds(0, n)])
        pltpu.sync_copy(wb.at[pl.ds(0, n)], out_hbm.at[pl.ds(out_row0 + off, n)])

    @pl.when(sid == NS - 1)
    def _():
        pltpu.sync_copy(aggs.at[pl.ds(_TAIL_ROW0, _TAIL_N)],
                        wb.at[pl.ds(0, _TAIL_N)])
        pltpu.sync_copy(wb.at[pl.ds(0, _TAIL_N)],
                        out_hbm.at[pl.ds(cid * N_NODES + _TAIL_ROW0, _TAIL_N)])


def _segment_msg_sum(nn, h, src, dst):
    mesh = plsc.VectorSubcoreMesh(core_axis_name="c", subcore_axis_name="s")
    f = pl.kernel(
        _sc_body,
        out_type=jax.ShapeDtypeStruct((NC * N_NODES, DIM), jnp.float32),
        mesh=mesh,
        scratch_types=(
            [
                pltpu.VMEM((EPW,), jnp.int32),
                pltpu.VMEM((EPW,), jnp.int32),
                pltpu.VMEM((WB, DIM), jnp.float32),
            ]
            + [pltpu.VMEM((EB, DIM), jnp.float32) for _ in range(NBUF)]
            + [pltpu.VMEM((EB, DIM), jnp.float32) for _ in range(NBUF)]
            + [pltpu.VMEM_SHARED((N_NODES, DIM), jnp.float32)]
            + [pltpu.SemaphoreType.DMA for _ in range(3 * NBUF)]
        ),
    )
    out = f(nn, h, src, dst)
    return out.reshape(NC, N_NODES, DIM)


def kernel(node_type, edge_index, distance, emb, W1s, Wc1s, bc1s, Wc2s, bc2s,
           W2s, b2s, W3s, b3s, Wa1, ba1, Wcls, bcls):
    src = edge_index[0]
    dst = edge_index[1]
    dist2 = distance.reshape(N_EDGES, 1)

    node = _embed(node_type, emb)
    for i in range(N_CONV):
        h = _edge_filter(dist2, Wc1s[i], bc1s[i], Wc2s[i], bc2s[i])
        nn = _new_node(node, W1s[i])
        agg2 = _segment_msg_sum(nn, h, src, dst)
        node = _update(node, agg2, W2s[i], b2s[i], W3s[i], b3s[i])
    return _head(node, Wa1, ba1, Wcls, bcls)


# speedup vs baseline: 2.9413x; 1.1378x over previous
import functools

import numpy as np
import jax
import jax.numpy as jnp
from jax import lax
from jax.experimental import pallas as pl
from jax.experimental.pallas import tpu as pltpu
from jax.experimental.pallas import tpu_sc as plsc

DIM = 128
N_NODES = 10000
N_EDGES = 320000
N_CONV = 3
CUTOFF = 5.0
N_CENTERS = 5
_CENTERS = [float(c) for c in np.linspace(0.0, CUTOFF, N_CENTERS)]
GAP = _CENTERS[1] - _CENTERS[0]
TYPE_NUM = 100

NC, NS = 2, 16
NW = NC * NS
EPW = N_EDGES // NW
EB = 16
NCHUNK = EPW // EB
RPS = 624
_ROW_CHUNKS = [(o, min(EB, RPS - o)) for o in range(0, RPS, EB)]
_TAIL_ROW0 = NS * RPS
_TAIL_N = N_NODES - _TAIL_ROW0

EB_H = 8000


def _sp05(x):
    z = 0.5 * x
    return jnp.where(z > 14.0, x, 2.0 * jnp.logaddexp(0.0, z))


def _shift_sp(x):
    return jnp.where(x > 20.0, x, jnp.logaddexp(0.0, x)) - float(np.log(2.0))


def _embed_body(nt_ref, emb_ref, out_ref):
    nt = nt_ref[...]
    ids = lax.broadcasted_iota(jnp.int32, (1, TYPE_NUM), 1)
    onehot = (nt == ids).astype(jnp.float32)
    out_ref[...] = jnp.dot(onehot, emb_ref[...],
                           preferred_element_type=jnp.float32)


def _embed(node_type, emb):
    return pl.pallas_call(
        _embed_body,
        out_shape=jax.ShapeDtypeStruct((N_NODES, DIM), jnp.float32),
    )(node_type.reshape(N_NODES, 1), emb)


def _h_body(d_ref, w1t_ref, b1_ref, w2t_ref, b2_ref, out_ref):
    d = d_ref[...]
    acc = jnp.broadcast_to(b1_ref[...], (EB_H, DIM))
    for k in range(N_CENTERS):
        r = jnp.exp((-1.0 / GAP) * (d - _CENTERS[k]) ** 2)
        acc = acc + r * w1t_ref[k][None, :]
    h = _sp05(acc)
    h = jnp.dot(h, w2t_ref[...], preferred_element_type=jnp.float32) + b2_ref[...]
    out_ref[...] = _sp05(h)


def _edge_filter(dist2, wc1, bc1, wc2, bc2):
    grid = (N_EDGES // EB_H,)
    return pl.pallas_call(
        _h_body,
        grid=grid,
        in_specs=[
            pl.BlockSpec((EB_H, 1), lambda i: (i, 0)),
            pl.BlockSpec((N_CENTERS, DIM), lambda i: (0, 0)),
            pl.BlockSpec((1, DIM), lambda i: (0, 0)),
            pl.BlockSpec((DIM, DIM), lambda i: (0, 0)),
            pl.BlockSpec((1, DIM), lambda i: (0, 0)),
        ],
        out_specs=pl.BlockSpec((EB_H, DIM), lambda i: (i, 0)),
        out_shape=jax.ShapeDtypeStruct((N_EDGES, DIM), jnp.float32),
    )(dist2, wc1.T, bc1.reshape(1, DIM), wc2.T, bc2.reshape(1, DIM))


def _mm_body(x_ref, wt_ref, out_ref):
    out_ref[...] = jnp.dot(x_ref[...], wt_ref[...],
                           preferred_element_type=jnp.float32)


def _new_node(node, w1):
    return pl.pallas_call(
        _mm_body,
        out_shape=jax.ShapeDtypeStruct((N_NODES, DIM), jnp.float32),
    )(node, w1.T)


def _update_body(node_ref, agg_ref, w2t_ref, b2_ref, w3t_ref, b3_ref, out_ref):
    agg = agg_ref[0] + agg_ref[1]
    cf1 = _sp05(jnp.dot(agg, w2t_ref[...],
                        preferred_element_type=jnp.float32) + b2_ref[...])
    out_ref[...] = (node_ref[...]
                    + jnp.dot(cf1, w3t_ref[...],
                              preferred_element_type=jnp.float32)
                    + b3_ref[...])


def _update(node, agg2, w2, b2, w3, b3):
    return pl.pallas_call(
        _update_body,
        out_shape=jax.ShapeDtypeStruct((N_NODES, DIM), jnp.float32),
    )(node, agg2, w2.T, b2.reshape(1, DIM), w3.T, b3.reshape(1, DIM))


def _head_body(node_ref, wa1t_ref, ba1_ref, wclst_ref, bcls_ref, out_ref):
    atom = jnp.dot(node_ref[...], wa1t_ref[...],
                   preferred_element_type=jnp.float32) + ba1_ref[...]
    res = _shift_sp(atom)
    out_ref[...] = jnp.dot(res, wclst_ref[...],
                           preferred_element_type=jnp.float32) + bcls_ref[...]


def _head(node, wa1, ba1, wcls, bcls):
    return pl.pallas_call(
        _head_body,
        out_shape=jax.ShapeDtypeStruct((N_NODES, TYPE_NUM), jnp.float32),
    )(node, wa1.T, ba1.reshape(1, 64), wcls.T, bcls.reshape(1, TYPE_NUM))


NBUF = 5
NGROUP = NCHUNK // NBUF
WB = 64
_WB_CHUNKS = [(o, min(WB, RPS - o)) for o in range(0, RPS, WB)]


def _sc_body(nn_hbm, h_hbm, src_hbm, dst_hbm, out_hbm,
             isrc, idst, wb, *rest):
    gvs = rest[0:NBUF]
    hvs = rest[NBUF:2 * NBUF]
    aggs = rest[2 * NBUF]
    hsems = rest[2 * NBUF + 1:2 * NBUF + 1 + NBUF]
    gsems = rest[2 * NBUF + 1 + NBUF:2 * NBUF + 1 + 2 * NBUF]
    ssems = rest[2 * NBUF + 1 + 2 * NBUF:2 * NBUF + 1 + 3 * NBUF]

    cid = lax.axis_index("c")
    sid = lax.axis_index("s")
    wid = cid * NS + sid
    base_e = wid * EPW

    pltpu.sync_copy(src_hbm.at[pl.ds(base_e, EPW)], isrc)
    pltpu.sync_copy(dst_hbm.at[pl.ds(base_e, EPW)], idst)

    zvec = jnp.zeros((16,), jnp.float32)

    def zrow(r, _):
        for j in range(DIM // 16):
            wb[r, pl.ds(j * 16, 16)] = zvec
        return 0

    lax.fori_loop(0, WB, zrow, 0)
    row0 = sid * RPS
    for off, n in _WB_CHUNKS:
        pltpu.sync_copy(wb.at[pl.ds(0, n)], aggs.at[pl.ds(row0 + off, n)])

    @pl.when(sid == NS - 1)
    def _():
        pltpu.sync_copy(wb.at[pl.ds(0, _TAIL_N)],
                        aggs.at[pl.ds(_TAIL_ROW0, _TAIL_N)])

    plsc.subcore_barrier()

    def group(g, _):
        c0 = g * NBUF
        in_descs = []
        for b in range(NBUF):
            c = c0 + b
            e0 = base_e + c * EB

            @pl.when(g > 0)
            def _(b=b, c=c):
                dreg = idst[pl.ds(c * EB, EB)]
                pltpu.make_async_copy(gvs[b], aggs.at[dreg], ssems[b]).wait()

            sreg = isrc[pl.ds(c * EB, EB)]
            hd = pltpu.async_copy(h_hbm.at[pl.ds(e0, EB)], hvs[b], hsems[b])
            gd = pltpu.async_copy(nn_hbm.at[sreg], gvs[b], gsems[b])
            in_descs.append((hd, gd))

        for b in range(NBUF):
            c = c0 + b
            hd, gd = in_descs[b]
            hd.wait()
            gd.wait()
            gvb, hvb = gvs[b], hvs[b]

            @plsc.parallel_loop(0, EB, 1, unroll=4)
            def _(r, gvb=gvb, hvb=hvb):
                for j in range(DIM // 16):
                    sl = pl.ds(j * 16, 16)
                    gvb[r, sl] = gvb[r, sl] * hvb[r, sl]
            dreg = idst[pl.ds(c * EB, EB)]
            pltpu.async_copy(gvb, aggs.at[dreg], ssems[b], add=True)
        return 0

    lax.fori_loop(0, NGROUP, group, 0)
    for b in range(NBUF):
        dreg = idst[pl.ds(((NGROUP - 1) * NBUF + b) * EB, EB)]
        pltpu.make_async_copy(gvs[b], aggs.at[dreg], ssems[b]).wait()
    plsc.subcore_barrier()

    out_row0 = cid * N_NODES + row0
    for off, n in _WB_CHUNKS:
        pltpu.sync_copy(aggs.at[pl.ds(row0 + off, n)], wb.at[pl.ds(0, n)])
        pltpu.sync_copy(wb.at[pl.ds(0, n)], out_hbm.at[pl.ds(out_row0 + off, n)])

    @pl.when(sid == NS - 1)
    def _():
        pltpu.sync_copy(aggs.at[pl.ds(_TAIL_ROW0, _TAIL_N)],
                        wb.at[pl.ds(0, _TAIL_N)])
        pltpu.sync_copy(wb.at[pl.ds(0, _TAIL_N)],
                        out_hbm.at[pl.ds(cid * N_NODES + _TAIL_ROW0, _TAIL_N)])


def _segment_msg_sum(nn, h, src, dst):
    mesh = plsc.VectorSubcoreMesh(core_axis_name="c", subcore_axis_name="s")
    f = pl.kernel(
        _sc_body,
        out_type=jax.ShapeDtypeStruct((NC * N_NODES, DIM), jnp.float32),
        mesh=mesh,
        scratch_types=(
            [
                pltpu.VMEM((EPW,), jnp.int32),
                pltpu.VMEM((EPW,), jnp.int32),
                pltpu.VMEM((WB, DIM), jnp.float32),
            ]
            + [pltpu.VMEM((EB, DIM), jnp.float32) for _ in range(NBUF)]
            + [pltpu.VMEM((EB, DIM), jnp.float32) for _ in range(NBUF)]
            + [pltpu.VMEM_SHARED((N_NODES, DIM), jnp.float32)]
            + [pltpu.SemaphoreType.DMA for _ in range(3 * NBUF)]
        ),
    )
    out = f(nn, h, src, dst)
    return out.reshape(NC, N_NODES, DIM)


def kernel(node_type, edge_index, distance, emb, W1s, Wc1s, bc1s, Wc2s, bc2s,
           W2s, b2s, W3s, b3s, Wa1, ba1, Wcls, bcls):
    src = edge_index[0]
    dst = edge_index[1]
    dist2 = distance.reshape(N_EDGES, 1)

    node = _embed(node_type, emb)
    for i in range(N_CONV):
        h = _edge_filter(dist2, Wc1s[i], bc1s[i], Wc2s[i], bc2s[i])
        nn = _new_node(node, W1s[i])
        agg2 = _segment_msg_sum(nn, h, src, dst)
        node = _update(node, agg2, W2s[i], b2s[i], W3s[i], b3s[i])
    return _head(node, Wa1, ba1, Wcls, bcls)
